# baseline (speedup 1.0000x reference)
import functools

import jax
import jax.numpy as jnp
from jax import lax
from jax.experimental import pallas as pl
from jax.experimental.pallas import tpu as pltpu
from jax.experimental.pallas import tpu_sc as plsc

VOCAB = 1000
EMB_D = 128
BATCH = 16384
CHUNK = 128
STAGE_ROWS = 64
STAGE_TAIL = VOCAB - 15 * STAGE_ROWS


@functools.cache
def _build():
    info = plsc.get_sparse_core_info()
    nc = info.num_cores
    nw = nc * info.num_subcores
    b_per_w = BATCH // nw
    n_chunks = b_per_w // CHUNK
    mesh = plsc.VectorSubcoreMesh(core_axis_name="c", subcore_axis_name="s")

    @functools.partial(
        pl.kernel,
        mesh=mesh,
        out_type=jax.ShapeDtypeStruct((BATCH, EMB_D), jnp.float32),
        scratch_types=[
            pltpu.VMEM((n_chunks, CHUNK), jnp.int32),
            pltpu.VMEM((b_per_w, EMB_D), jnp.float32),
            pltpu.VMEM_SHARED((VOCAB, EMB_D), jnp.float32),
            pltpu.SemaphoreType.DMA,
            pltpu.SemaphoreType.DMA,
            pltpu.SemaphoreType.DMA,
            pltpu.SemaphoreType.DMA,
            pltpu.SemaphoreType.DMA,
            pltpu.SemaphoreType.DMA,
        ],
    )
    def emb_kernel(
        idx_hbm, table_hbm, out_hbm, idx_v, rows_v, table_sp,
        g0, g1, g2, g3, tsem, ssem
    ):
        gsems = [g0, g1, g2, g3]
        sid = lax.axis_index("s")
        wid = sid * nc + lax.axis_index("c")
        base = wid * b_per_w

        @pl.when(sid < 15)
        def _stage():
            c = pltpu.async_copy(
                table_hbm.at[pl.ds(sid * STAGE_ROWS, STAGE_ROWS)],
                table_sp.at[pl.ds(sid * STAGE_ROWS, STAGE_ROWS)],
                tsem,
            )
            pltpu.sync_copy(idx_hbm.at[wid], idx_v)
            c.wait()

        @pl.when(sid == 15)
        def _stage_tail():
            c = pltpu.async_copy(
                table_hbm.at[pl.ds(15 * STAGE_ROWS, STAGE_TAIL)],
                table_sp.at[pl.ds(15 * STAGE_ROWS, STAGE_TAIL)],
                tsem,
            )
            pltpu.sync_copy(idx_hbm.at[wid], idx_v)
            c.wait()

        plsc.subcore_barrier()

        gathers = []
        for j in range(n_chunks):
            gathers.append(
                pltpu.async_copy(
                    table_sp.at[idx_v.at[j]],
                    rows_v.at[pl.ds(j * CHUNK, CHUNK)],
                    gsems[j],
                )
            )

        stores = []
        for j in range(n_chunks):
            gathers[j].wait()
            stores.append(
                pltpu.async_copy(
                    rows_v.at[pl.ds(j * CHUNK, CHUNK)],
                    out_hbm.at[pl.ds(base + j * CHUNK, CHUNK)],
                    ssem,
                )
            )
        for s in stores:
            s.wait()

    return emb_kernel, nw, n_chunks


def kernel(x, table):
    emb_kernel, nw, n_chunks = _build()
    idx = x.astype(jnp.int32).reshape(nw, n_chunks, CHUNK)
    return emb_kernel(idx, table)


# speedup vs baseline: 1.0160x; 1.0160x over previous
import functools

import jax
import jax.numpy as jnp
from jax import lax
from jax.experimental import pallas as pl
from jax.experimental.pallas import tpu as pltpu
from jax.experimental.pallas import tpu_sc as plsc

VOCAB = 1000
EMB_D = 128
BATCH = 16384
CHUNK = 64
STAGE_ROWS = 64
STAGE_TAIL = VOCAB - 15 * STAGE_ROWS


@functools.cache
def _build():
    info = plsc.get_sparse_core_info()
    nc = info.num_cores
    nw = nc * info.num_subcores
    b_per_w = BATCH // nw
    n_chunks = b_per_w // CHUNK
    mesh = plsc.VectorSubcoreMesh(core_axis_name="c", subcore_axis_name="s")

    @functools.partial(
        pl.kernel,
        mesh=mesh,
        out_type=jax.ShapeDtypeStruct((BATCH, EMB_D), jnp.float32),
        scratch_types=[
            pltpu.VMEM((n_chunks, CHUNK), jnp.int32),
            pltpu.VMEM((b_per_w, EMB_D), jnp.float32),
            pltpu.VMEM_SHARED((VOCAB, EMB_D), jnp.float32),
            pltpu.SemaphoreType.DMA,
            pltpu.SemaphoreType.DMA,
            pltpu.SemaphoreType.DMA,
            pltpu.SemaphoreType.DMA,
            pltpu.SemaphoreType.DMA,
            pltpu.SemaphoreType.DMA,
            pltpu.SemaphoreType.DMA,
            pltpu.SemaphoreType.DMA,
            pltpu.SemaphoreType.DMA,
            pltpu.SemaphoreType.DMA,
        ],
    )
    def emb_kernel(
        idx_hbm, table_hbm, out_hbm, idx_v, rows_v, table_sp,
        g0, g1, g2, g3, g4, g5, g6, g7, tsem, ssem
    ):
        gsems = [g0, g1, g2, g3, g4, g5, g6, g7]
        sid = lax.axis_index("s")
        wid = sid * nc + lax.axis_index("c")
        base = wid * b_per_w

        @pl.when(sid < 15)
        def _stage():
            c = pltpu.async_copy(
                table_hbm.at[pl.ds(sid * STAGE_ROWS, STAGE_ROWS)],
                table_sp.at[pl.ds(sid * STAGE_ROWS, STAGE_ROWS)],
                tsem,
            )
            pltpu.sync_copy(idx_hbm.at[wid], idx_v)
            c.wait()

        @pl.when(sid == 15)
        def _stage_tail():
            c = pltpu.async_copy(
                table_hbm.at[pl.ds(15 * STAGE_ROWS, STAGE_TAIL)],
                table_sp.at[pl.ds(15 * STAGE_ROWS, STAGE_TAIL)],
                tsem,
            )
            pltpu.sync_copy(idx_hbm.at[wid], idx_v)
            c.wait()

        plsc.subcore_barrier()

        gathers = []
        for j in range(n_chunks):
            gathers.append(
                pltpu.async_copy(
                    table_sp.at[idx_v.at[j]],
                    rows_v.at[pl.ds(j * CHUNK, CHUNK)],
                    gsems[j],
                )
            )

        stores = []
        for j in range(n_chunks):
            gathers[j].wait()
            stores.append(
                pltpu.async_copy(
                    rows_v.at[pl.ds(j * CHUNK, CHUNK)],
                    out_hbm.at[pl.ds(base + j * CHUNK, CHUNK)],
                    ssem,
                )
            )
        for s in stores:
            s.wait()

    return emb_kernel, nw, n_chunks


def kernel(x, table):
    emb_kernel, nw, n_chunks = _build()
    idx = x.astype(jnp.int32).reshape(nw, n_chunks, CHUNK)
    return emb_kernel(idx, table)
